# baseline (speedup 1.0000x reference)
import functools

import jax
import jax.numpy as jnp
import numpy as np
from jax import lax
from jax.experimental import pallas as pl
from jax.experimental.pallas import tpu as pltpu
from jax.experimental.pallas import tpu_sc as plsc

N_NODES = 10000
N_EDGES = 160000
D_IN = 256
D_HID = 256
N_CLASSES = 40
N_EXPERTS = 8

NUM_SC_CORES = 2
NUM_SUBCORES = 16
EDGES_PER_TILE = N_EDGES // NUM_SUBCORES
EDGE_BATCH = 80
BATCHES_PER_TILE = EDGES_PER_TILE // EDGE_BATCH
ROWS_PER_TILE = 624
ROWS_LAST_TILE = N_NODES - ROWS_PER_TILE * (NUM_SUBCORES - 1)

EC = N_EXPERTS * N_CLASSES

NODE_BLOCK = 1000
NUM_NODE_BLOCKS = N_NODES // NODE_BLOCK


@functools.cache
def _mesh():
    return plsc.VectorSubcoreMesh(
        core_axis_name="c", subcore_axis_name="s",
        num_cores=NUM_SC_CORES, num_subcores=NUM_SUBCORES)


DH = 128


def _zero_my_rows(zer_hbm, acc_sh, s):
    row0 = s * ROWS_PER_TILE

    @pl.when(s < NUM_SUBCORES - 1)
    def _():
        pltpu.sync_copy(zer_hbm.at[pl.ds(row0, ROWS_PER_TILE)],
                        acc_sh.at[pl.ds(row0, ROWS_PER_TILE)])

    @pl.when(s == NUM_SUBCORES - 1)
    def _():
        pltpu.sync_copy(zer_hbm.at[pl.ds(row0, ROWS_LAST_TILE)],
                        acc_sh.at[pl.ds(row0, ROWS_LAST_TILE)])


def _writeback_my_rows(acc_sh, out, s):
    row0 = s * ROWS_PER_TILE

    @pl.when(s < NUM_SUBCORES - 1)
    def _():
        pltpu.sync_copy(acc_sh.at[pl.ds(row0, ROWS_PER_TILE)],
                        out.at[pl.ds(row0, ROWS_PER_TILE)])

    @pl.when(s == NUM_SUBCORES - 1)
    def _():
        pltpu.sync_copy(acc_sh.at[pl.ds(row0, ROWS_LAST_TILE)],
                        out.at[pl.ds(row0, ROWS_LAST_TILE)])


EDGES_PER_WORKER = N_EDGES // (NUM_SC_CORES * NUM_SUBCORES)
MAIN_BATCH = 80
N_MAIN = 62
TAIL_BATCH = EDGES_PER_WORKER - N_MAIN * MAIN_BATCH


@functools.cache
def _build_segsum():
    @functools.partial(
        pl.kernel,
        out_type=(
            jax.ShapeDtypeStruct((N_NODES, DH), jnp.float32),
            jax.ShapeDtypeStruct((N_NODES, DH), jnp.float32),
        ),
        mesh=_mesh(),
        scratch_types=[
            pltpu.VMEM((N_MAIN, MAIN_BATCH), jnp.int32),
            pltpu.VMEM((N_MAIN, MAIN_BATCH), jnp.int32),
            pltpu.VMEM((1, TAIL_BATCH), jnp.int32),
            pltpu.VMEM((1, TAIL_BATCH), jnp.int32),
            pltpu.VMEM((MAIN_BATCH, DH), jnp.float32),
            pltpu.VMEM((MAIN_BATCH, DH), jnp.float32),
            pltpu.VMEM((MAIN_BATCH, DH), jnp.float32),
            pltpu.VMEM_SHARED((N_NODES, DH), jnp.float32),
            pltpu.SemaphoreType.DMA,
            pltpu.SemaphoreType.DMA,
            pltpu.SemaphoreType.DMA,
            pltpu.SemaphoreType.DMA,
        ],
    )
    def segsum(h, srcm_hbm, dstm_hbm, srct_hbm, dstt_hbm, zer_hbm,
               outa, outb,
               srcm_v, dstm_v, srct_v, dstt_v, r0, r1, r2, acc_sh,
               g0, g1, g2, ss):
        c = lax.axis_index("c")
        s = lax.axis_index("s")
        bufs = (r0, r1, r2)
        gsems = (g0, g1, g2)

        pltpu.sync_copy(srcm_hbm.at[c, s], srcm_v)
        d1 = pltpu.async_copy(dstm_hbm.at[c, s], dstm_v, ss)
        d2 = pltpu.async_copy(srct_hbm.at[c, s], srct_v, ss)
        d3 = pltpu.async_copy(dstt_hbm.at[c, s], dstt_v, ss)

        pltpu.async_copy(h.at[srcm_v.at[0]], r0, g0)
        _zero_my_rows(zer_hbm, acc_sh, s)
        d1.wait()
        d2.wait()
        d3.wait()
        plsc.subcore_barrier()

        def step(j, cur, gcur, nxt, gnxt):
            pltpu.make_async_copy(h.at[srcm_v.at[j]], cur, gcur).wait()

            @pl.when(j >= 2)
            def _():
                pltpu.make_async_copy(
                    nxt, acc_sh.at[dstm_v.at[j]], ss).wait()

            @pl.when(j < N_MAIN - 1)
            def _():
                pltpu.async_copy(h.at[srcm_v.at[j + 1]], nxt, gnxt)

            pltpu.async_copy(cur, acc_sh.at[dstm_v.at[j]], ss, add=True)

        def body(j, carry):
            @pl.when(j % 3 == 0)
            def _():
                step(j, r0, g0, r1, g1)

            @pl.when(j % 3 == 1)
            def _():
                step(j, r1, g1, r2, g2)

            @pl.when(j % 3 == 2)
            def _():
                step(j, r2, g2, r0, g0)

            return carry

        lax.fori_loop(0, N_MAIN, body, 0)

        pltpu.make_async_copy(
            bufs[(N_MAIN - 2) % 3],
            acc_sh.at[dstm_v.at[N_MAIN - 2]], ss).wait()
        pltpu.make_async_copy(
            bufs[(N_MAIN - 1) % 3],
            acc_sh.at[dstm_v.at[N_MAIN - 1]], ss).wait()
        rtail = r0.at[pl.ds(0, TAIL_BATCH)]
        pltpu.async_copy(h.at[srct_v.at[0]], rtail, g0).wait()
        pltpu.sync_copy(rtail, acc_sh.at[dstt_v.at[0]], add=True)
        plsc.subcore_barrier()

        @pl.when(c == 0)
        def _():
            _writeback_my_rows(acc_sh, outa, s)

        @pl.when(c == 1)
        def _():
            _writeback_my_rows(acc_sh, outb, s)

    return segsum


def _segsum(h, srcm, dstm, srct, dstt, zer):
    return _build_segsum()(h, srcm, dstm, srct, dstt, zer)


def _tc_experts_body(x_ref, a0a_ref, a0b_ref, a1a_ref, a1b_ref, noise_ref,
                     wg_ref, wn_ref, thr_ref, w1_ref, w2_ref,
                     u0_ref, u1_ref, u2_ref, gates_ref):
    x = x_ref[...]
    z = x + jnp.concatenate([a0a_ref[...] + a0b_ref[...],
                             a1a_ref[...] + a1b_ref[...]], axis=1)

    clean = jnp.dot(x, wg_ref[...], preferred_element_type=jnp.float32)
    t = jnp.dot(x, wn_ref[...], preferred_element_type=jnp.float32)
    std = jnp.log1p(jnp.exp(-jnp.abs(t))) + jnp.maximum(t, 0.0) + 1e-2
    noisy = clean + noise_ref[...] * std
    scores = noisy - thr_ref[...]
    open_mask = (scores > 0.0).astype(jnp.float32)
    m = jnp.max(noisy, axis=1, keepdims=True)
    ex = jnp.exp(noisy - m)
    sm = ex / jnp.sum(ex, axis=1, keepdims=True)
    raw = sm * open_mask
    gates_ref[...] = raw / (jnp.sum(raw, axis=1, keepdims=True) + 1e-9)

    h = jnp.maximum(
        jnp.dot(z.astype(jnp.bfloat16), w1_ref[...],
                preferred_element_type=jnp.float32), 0.0)
    u = jnp.dot(h.astype(jnp.bfloat16), w2_ref[...],
                preferred_element_type=jnp.float32)
    u0_ref[...] = u[:, :DH]
    u1_ref[...] = u[:, DH:2 * DH]
    u2_ref[...] = u[:, 2 * DH:]


def _tc_experts(x, a0a, a0b, a1a, a1b, noise, w_gate, w_noise, thr,
                w1cat, w2bd):
    blk = lambda shape: pl.BlockSpec(shape, lambda i: (0, 0))
    return pl.pallas_call(
        _tc_experts_body,
        grid=(NUM_NODE_BLOCKS,),
        in_specs=[
            pl.BlockSpec((NODE_BLOCK, D_IN), lambda i: (i, 0)),
            pl.BlockSpec((NODE_BLOCK, DH), lambda i: (i, 0)),
            pl.BlockSpec((NODE_BLOCK, DH), lambda i: (i, 0)),
            pl.BlockSpec((NODE_BLOCK, DH), lambda i: (i, 0)),
            pl.BlockSpec((NODE_BLOCK, DH), lambda i: (i, 0)),
            pl.BlockSpec((NODE_BLOCK, N_EXPERTS), lambda i: (i, 0)),
            blk((D_IN, N_EXPERTS)),
            blk((D_IN, N_EXPERTS)),
            blk((1, N_EXPERTS)),
            blk((D_IN, N_EXPERTS * D_HID)),
            blk((N_EXPERTS * D_HID, 3 * DH)),
        ],
        out_specs=[
            pl.BlockSpec((NODE_BLOCK, DH), lambda i: (i, 0)),
            pl.BlockSpec((NODE_BLOCK, DH), lambda i: (i, 0)),
            pl.BlockSpec((NODE_BLOCK, DH), lambda i: (i, 0)),
            pl.BlockSpec((NODE_BLOCK, N_EXPERTS), lambda i: (i, 0)),
        ],
        out_shape=[
            jax.ShapeDtypeStruct((N_NODES, DH), jnp.float32),
            jax.ShapeDtypeStruct((N_NODES, DH), jnp.float32),
            jax.ShapeDtypeStruct((N_NODES, DH), jnp.float32),
            jax.ShapeDtypeStruct((N_NODES, N_EXPERTS), jnp.float32),
        ],
    )(x, a0a, a0b, a1a, a1b, noise, w_gate, w_noise, thr, w1cat, w2bd)


_W2_PLACE = np.zeros((N_EXPERTS, N_CLASSES, 3 * 128), np.float32)
for _e in range(N_EXPERTS):
    for _c in range(N_CLASSES):
        _W2_PLACE[_e, _c, 40 * _e + _c] = 1.0
_W2_PLACE = _W2_PLACE.astype(np.float32)

_REP_FULL = np.kron(np.eye(N_EXPERTS), np.ones((1, N_CLASSES))).astype(np.float32)
_SEL_FULL = np.kron(np.ones((N_EXPERTS, 1)), np.eye(N_CLASSES)).astype(np.float32)


_REP_PAD = np.concatenate(
    [_REP_FULL, np.zeros((N_EXPERTS, 3 * DH - EC), np.float32)], axis=1)
_SEL_PAD = np.concatenate(
    [_SEL_FULL, np.zeros((3 * DH - EC, N_CLASSES), np.float32)], axis=0)


def _tc_combine_body(u0_ref, u1_ref, u2_ref, p0a_ref, p0b_ref, p1a_ref,
                     p1b_ref, p2a_ref, p2b_ref, gates_ref, repp_ref,
                     selp_ref, y_ref):
    g = gates_ref[...]
    su = jnp.concatenate(
        [u0_ref[...] + p0a_ref[...] + p0b_ref[...],
         u1_ref[...] + p1a_ref[...] + p1b_ref[...],
         u2_ref[...] + p2a_ref[...] + p2b_ref[...]], axis=1)
    gp = jnp.dot(g, repp_ref[...], preferred_element_type=jnp.float32)
    y_ref[...] = jnp.dot(su * gp, selp_ref[...],
                         preferred_element_type=jnp.float32)


def _tc_combine(u0, u1, u2, p0a, p0b, p1a, p1b, p2a, p2b, gates):
    blk = lambda shape: pl.BlockSpec(shape, lambda i: (0, 0))
    consts = (jnp.asarray(_REP_PAD), jnp.asarray(_SEL_PAD))
    nb = lambda: pl.BlockSpec((NODE_BLOCK, DH), lambda i: (i, 0))
    return pl.pallas_call(
        _tc_combine_body,
        grid=(NUM_NODE_BLOCKS,),
        in_specs=[
            nb(), nb(), nb(), nb(), nb(), nb(), nb(), nb(), nb(),
            pl.BlockSpec((NODE_BLOCK, N_EXPERTS), lambda i: (i, 0)),
            blk((N_EXPERTS, 3 * DH)), blk((3 * DH, N_CLASSES)),
        ],
        out_specs=pl.BlockSpec((NODE_BLOCK, N_CLASSES), lambda i: (i, 0)),
        out_shape=jax.ShapeDtypeStruct((N_NODES, N_CLASSES), jnp.float32),
    )(u0, u1, u2, p0a, p0b, p1a, p1b, p2a, p2b, gates, *consts)


def kernel(x, edge_index, noise, w_gate, w_noise, gate_threshold, W1, W2):
    ncut = N_MAIN * MAIN_BATCH
    src = edge_index[0].astype(jnp.int32).reshape(
        NUM_SC_CORES, NUM_SUBCORES, EDGES_PER_WORKER)
    dst = edge_index[1].astype(jnp.int32).reshape(
        NUM_SC_CORES, NUM_SUBCORES, EDGES_PER_WORKER)
    srcm = src[:, :, :ncut].reshape(NUM_SC_CORES, NUM_SUBCORES,
                                    N_MAIN, MAIN_BATCH)
    dstm = dst[:, :, :ncut].reshape(NUM_SC_CORES, NUM_SUBCORES,
                                    N_MAIN, MAIN_BATCH)
    srct = src[:, :, ncut:].reshape(NUM_SC_CORES, NUM_SUBCORES,
                                    1, TAIL_BATCH)
    dstt = dst[:, :, ncut:].reshape(NUM_SC_CORES, NUM_SUBCORES,
                                    1, TAIL_BATCH)
    idx = (srcm, dstm, srct, dstt)

    zer = jnp.zeros((N_NODES, DH), jnp.float32)
    x0 = x[:, :DH]
    x1 = x[:, DH:]
    x0a, x0b = _segsum(x0, *idx, zer)
    x1a, x1b = _segsum(x1, *idx, zer)

    w1cat = jnp.transpose(W1, (1, 0, 2)).reshape(
        D_IN, N_EXPERTS * D_HID).astype(jnp.bfloat16)
    eyeb = jnp.asarray(_W2_PLACE)
    w2s = jax.lax.dot_general(
        W2, eyeb, (((2,), (1,)), ((0,), (0,)))).reshape(
            N_EXPERTS * D_HID, 3 * DH).astype(jnp.bfloat16)
    thr = gate_threshold.reshape(1, N_EXPERTS)

    u0, u1, u2, gates = _tc_experts(x, x0a, x0b, x1a, x1b, noise, w_gate,
                                    w_noise, thr, w1cat, w2s)

    p0a, p0b = _segsum(u0, *idx, zer)
    p1a, p1b = _segsum(u1, *idx, zer)
    p2a, p2b = _segsum(u2, *idx, zer)

    return _tc_combine(u0, u1, u2, p0a, p0b, p1a, p1b, p2a, p2b, gates)


# speedup vs baseline: 1.3869x; 1.3869x over previous
import functools

import jax
import jax.numpy as jnp
import numpy as np
from jax import lax
from jax.experimental import pallas as pl
from jax.experimental.pallas import tpu as pltpu
from jax.experimental.pallas import tpu_sc as plsc

N_NODES = 10000
N_EDGES = 160000
D_IN = 256
D_HID = 256
N_CLASSES = 40
N_EXPERTS = 8

NUM_SC_CORES = 2
NUM_SUBCORES = 16
EDGES_PER_TILE = N_EDGES // NUM_SUBCORES
EDGE_BATCH = 80
BATCHES_PER_TILE = EDGES_PER_TILE // EDGE_BATCH
ROWS_PER_TILE = 624
ROWS_LAST_TILE = N_NODES - ROWS_PER_TILE * (NUM_SUBCORES - 1)

EC = N_EXPERTS * N_CLASSES

NODE_BLOCK = 1000
NUM_NODE_BLOCKS = N_NODES // NODE_BLOCK


@functools.cache
def _mesh():
    return plsc.VectorSubcoreMesh(
        core_axis_name="c", subcore_axis_name="s",
        num_cores=NUM_SC_CORES, num_subcores=NUM_SUBCORES)


DH = 128


def _zero_my_rows(zer_hbm, acc_sh, s):
    row0 = s * ROWS_PER_TILE

    @pl.when(s < NUM_SUBCORES - 1)
    def _():
        pltpu.sync_copy(zer_hbm.at[pl.ds(row0, ROWS_PER_TILE)],
                        acc_sh.at[pl.ds(row0, ROWS_PER_TILE)])

    @pl.when(s == NUM_SUBCORES - 1)
    def _():
        pltpu.sync_copy(zer_hbm.at[pl.ds(row0, ROWS_LAST_TILE)],
                        acc_sh.at[pl.ds(row0, ROWS_LAST_TILE)])


def _writeback_my_rows(acc_sh, out, s):
    row0 = s * ROWS_PER_TILE

    @pl.when(s < NUM_SUBCORES - 1)
    def _():
        pltpu.sync_copy(acc_sh.at[pl.ds(row0, ROWS_PER_TILE)],
                        out.at[pl.ds(row0, ROWS_PER_TILE)])

    @pl.when(s == NUM_SUBCORES - 1)
    def _():
        pltpu.sync_copy(acc_sh.at[pl.ds(row0, ROWS_LAST_TILE)],
                        out.at[pl.ds(row0, ROWS_LAST_TILE)])


EDGES_PER_WORKER = N_EDGES // (NUM_SC_CORES * NUM_SUBCORES)
MAIN_BATCH = 80
N_MAIN = 62
TAIL_BATCH = EDGES_PER_WORKER - N_MAIN * MAIN_BATCH


@functools.cache
def _build_segsum():
    @functools.partial(
        pl.kernel,
        out_type=(
            jax.ShapeDtypeStruct((N_NODES, DH), jnp.float32),
            jax.ShapeDtypeStruct((N_NODES, DH), jnp.float32),
        ),
        mesh=_mesh(),
        scratch_types=[
            pltpu.VMEM((N_MAIN, MAIN_BATCH), jnp.int32),
            pltpu.VMEM((N_MAIN, MAIN_BATCH), jnp.int32),
            pltpu.VMEM((1, TAIL_BATCH), jnp.int32),
            pltpu.VMEM((1, TAIL_BATCH), jnp.int32),
            pltpu.VMEM((MAIN_BATCH, DH), jnp.float32),
            pltpu.VMEM((MAIN_BATCH, DH), jnp.float32),
            pltpu.VMEM((MAIN_BATCH, DH), jnp.float32),
            pltpu.VMEM_SHARED((N_NODES, DH), jnp.float32),
            pltpu.SemaphoreType.DMA,
            pltpu.SemaphoreType.DMA,
            pltpu.SemaphoreType.DMA,
            pltpu.SemaphoreType.DMA,
        ],
    )
    def segsum(h, srcm_hbm, dstm_hbm, srct_hbm, dstt_hbm, zer_hbm,
               outa, outb,
               srcm_v, dstm_v, srct_v, dstt_v, r0, r1, r2, acc_sh,
               g0, g1, g2, ss):
        c = lax.axis_index("c")
        s = lax.axis_index("s")
        bufs = (r0, r1, r2)
        gsems = (g0, g1, g2)

        pltpu.sync_copy(srcm_hbm.at[c, s], srcm_v)
        d1 = pltpu.async_copy(dstm_hbm.at[c, s], dstm_v, ss)
        d2 = pltpu.async_copy(srct_hbm.at[c, s], srct_v, ss)
        d3 = pltpu.async_copy(dstt_hbm.at[c, s], dstt_v, ss)

        pltpu.async_copy(h.at[srcm_v.at[0]], r0, g0)
        pltpu.async_copy(h.at[srcm_v.at[1]], r1, g1)
        _zero_my_rows(zer_hbm, acc_sh, s)
        d1.wait()
        d2.wait()
        d3.wait()
        plsc.subcore_barrier()

        def step(j, cur, gcur, prv, tw0):
            pltpu.make_async_copy(h.at[srcm_v.at[j]], cur, gcur).wait()

            @pl.when(j >= 1)
            def _():
                pltpu.make_async_copy(
                    prv, acc_sh.at[dstm_v.at[j - 1]], ss).wait()

            @pl.when(j < N_MAIN - 2)
            def _():
                pltpu.async_copy(h.at[srcm_v.at[j + 2]], prv, tw0)

            pltpu.async_copy(cur, acc_sh.at[dstm_v.at[j]], ss, add=True)

        def body(j, carry):
            @pl.when(j % 3 == 0)
            def _():
                step(j, r0, g0, r2, g2)

            @pl.when(j % 3 == 1)
            def _():
                step(j, r1, g1, r0, g0)

            @pl.when(j % 3 == 2)
            def _():
                step(j, r2, g2, r1, g1)

            return carry

        lax.fori_loop(0, N_MAIN, body, 0)

        pltpu.make_async_copy(
            bufs[(N_MAIN - 1) % 3],
            acc_sh.at[dstm_v.at[N_MAIN - 1]], ss).wait()
        rtail = r0.at[pl.ds(0, TAIL_BATCH)]
        pltpu.async_copy(h.at[srct_v.at[0]], rtail, g0).wait()
        pltpu.sync_copy(rtail, acc_sh.at[dstt_v.at[0]], add=True)
        plsc.subcore_barrier()

        @pl.when(c == 0)
        def _():
            _writeback_my_rows(acc_sh, outa, s)

        @pl.when(c == 1)
        def _():
            _writeback_my_rows(acc_sh, outb, s)

    return segsum


def _segsum(h, srcm, dstm, srct, dstt, zer):
    return _build_segsum()(h, srcm, dstm, srct, dstt, zer)


def _tc_experts_body(x_ref, a0a_ref, a0b_ref, a1a_ref, a1b_ref, noise_ref,
                     wg_ref, wn_ref, thr_ref, w1_ref, w2_ref,
                     u0_ref, u1_ref, u2_ref, gates_ref):
    x = x_ref[...]
    z = x + jnp.concatenate([a0a_ref[...] + a0b_ref[...],
                             a1a_ref[...] + a1b_ref[...]], axis=1)

    clean = jnp.dot(x, wg_ref[...], preferred_element_type=jnp.float32)
    t = jnp.dot(x, wn_ref[...], preferred_element_type=jnp.float32)
    std = jnp.log1p(jnp.exp(-jnp.abs(t))) + jnp.maximum(t, 0.0) + 1e-2
    noisy = clean + noise_ref[...] * std
    scores = noisy - thr_ref[...]
    open_mask = (scores > 0.0).astype(jnp.float32)
    m = jnp.max(noisy, axis=1, keepdims=True)
    ex = jnp.exp(noisy - m)
    sm = ex / jnp.sum(ex, axis=1, keepdims=True)
    raw = sm * open_mask
    gates_ref[...] = raw / (jnp.sum(raw, axis=1, keepdims=True) + 1e-9)

    h = jnp.maximum(
        jnp.dot(z.astype(jnp.bfloat16), w1_ref[...],
                preferred_element_type=jnp.float32), 0.0)
    hb = h.astype(jnp.bfloat16)
    us = [jnp.dot(hb[:, e * D_HID:(e + 1) * D_HID], w2_ref[e],
                  preferred_element_type=jnp.float32)
          for e in range(N_EXPERTS)]
    u = jnp.concatenate(
        us + [jnp.zeros((NODE_BLOCK, 3 * DH - EC), jnp.float32)], axis=1)
    u0_ref[...] = u[:, :DH]
    u1_ref[...] = u[:, DH:2 * DH]
    u2_ref[...] = u[:, 2 * DH:]


def _tc_experts(x, a0a, a0b, a1a, a1b, noise, w_gate, w_noise, thr,
                w1cat, w2bd):
    blk = lambda shape: pl.BlockSpec(shape, lambda i: (0, 0))
    return pl.pallas_call(
        _tc_experts_body,
        grid=(NUM_NODE_BLOCKS,),
        in_specs=[
            pl.BlockSpec((NODE_BLOCK, D_IN), lambda i: (i, 0)),
            pl.BlockSpec((NODE_BLOCK, DH), lambda i: (i, 0)),
            pl.BlockSpec((NODE_BLOCK, DH), lambda i: (i, 0)),
            pl.BlockSpec((NODE_BLOCK, DH), lambda i: (i, 0)),
            pl.BlockSpec((NODE_BLOCK, DH), lambda i: (i, 0)),
            pl.BlockSpec((NODE_BLOCK, N_EXPERTS), lambda i: (i, 0)),
            blk((D_IN, N_EXPERTS)),
            blk((D_IN, N_EXPERTS)),
            blk((1, N_EXPERTS)),
            blk((D_IN, N_EXPERTS * D_HID)),
            pl.BlockSpec((N_EXPERTS, D_HID, N_CLASSES), lambda i: (0, 0, 0)),
        ],
        out_specs=[
            pl.BlockSpec((NODE_BLOCK, DH), lambda i: (i, 0)),
            pl.BlockSpec((NODE_BLOCK, DH), lambda i: (i, 0)),
            pl.BlockSpec((NODE_BLOCK, DH), lambda i: (i, 0)),
            pl.BlockSpec((NODE_BLOCK, N_EXPERTS), lambda i: (i, 0)),
        ],
        out_shape=[
            jax.ShapeDtypeStruct((N_NODES, DH), jnp.float32),
            jax.ShapeDtypeStruct((N_NODES, DH), jnp.float32),
            jax.ShapeDtypeStruct((N_NODES, DH), jnp.float32),
            jax.ShapeDtypeStruct((N_NODES, N_EXPERTS), jnp.float32),
        ],
    )(x, a0a, a0b, a1a, a1b, noise, w_gate, w_noise, thr, w1cat, w2bd)


_REP_FULL = np.kron(np.eye(N_EXPERTS), np.ones((1, N_CLASSES))).astype(np.float32)
_SEL_FULL = np.kron(np.ones((N_EXPERTS, 1)), np.eye(N_CLASSES)).astype(np.float32)


_REP_PAD = np.concatenate(
    [_REP_FULL, np.zeros((N_EXPERTS, 3 * DH - EC), np.float32)], axis=1)
_SEL_PAD = np.concatenate(
    [_SEL_FULL, np.zeros((3 * DH - EC, N_CLASSES), np.float32)], axis=0)


def _tc_combine_body(u0_ref, u1_ref, u2_ref, p0a_ref, p0b_ref, p1a_ref,
                     p1b_ref, p2a_ref, p2b_ref, gates_ref, repp_ref,
                     selp_ref, y_ref):
    g = gates_ref[...]
    su = jnp.concatenate(
        [u0_ref[...] + p0a_ref[...] + p0b_ref[...],
         u1_ref[...] + p1a_ref[...] + p1b_ref[...],
         u2_ref[...] + p2a_ref[...] + p2b_ref[...]], axis=1)
    gp = jnp.dot(g, repp_ref[...], preferred_element_type=jnp.float32)
    y_ref[...] = jnp.dot(su * gp, selp_ref[...],
                         preferred_element_type=jnp.float32)


def _tc_combine(u0, u1, u2, p0a, p0b, p1a, p1b, p2a, p2b, gates):
    blk = lambda shape: pl.BlockSpec(shape, lambda i: (0, 0))
    consts = (jnp.asarray(_REP_PAD), jnp.asarray(_SEL_PAD))
    nb = lambda: pl.BlockSpec((NODE_BLOCK, DH), lambda i: (i, 0))
    return pl.pallas_call(
        _tc_combine_body,
        grid=(NUM_NODE_BLOCKS,),
        in_specs=[
            nb(), nb(), nb(), nb(), nb(), nb(), nb(), nb(), nb(),
            pl.BlockSpec((NODE_BLOCK, N_EXPERTS), lambda i: (i, 0)),
            blk((N_EXPERTS, 3 * DH)), blk((3 * DH, N_CLASSES)),
        ],
        out_specs=pl.BlockSpec((NODE_BLOCK, N_CLASSES), lambda i: (i, 0)),
        out_shape=jax.ShapeDtypeStruct((N_NODES, N_CLASSES), jnp.float32),
    )(u0, u1, u2, p0a, p0b, p1a, p1b, p2a, p2b, gates, *consts)


def kernel(x, edge_index, noise, w_gate, w_noise, gate_threshold, W1, W2):
    ncut = N_MAIN * MAIN_BATCH
    src = edge_index[0].astype(jnp.int32).reshape(
        NUM_SC_CORES, NUM_SUBCORES, EDGES_PER_WORKER)
    dst = edge_index[1].astype(jnp.int32).reshape(
        NUM_SC_CORES, NUM_SUBCORES, EDGES_PER_WORKER)
    srcm = src[:, :, :ncut].reshape(NUM_SC_CORES, NUM_SUBCORES,
                                    N_MAIN, MAIN_BATCH)
    dstm = dst[:, :, :ncut].reshape(NUM_SC_CORES, NUM_SUBCORES,
                                    N_MAIN, MAIN_BATCH)
    srct = src[:, :, ncut:].reshape(NUM_SC_CORES, NUM_SUBCORES,
                                    1, TAIL_BATCH)
    dstt = dst[:, :, ncut:].reshape(NUM_SC_CORES, NUM_SUBCORES,
                                    1, TAIL_BATCH)
    idx = (srcm, dstm, srct, dstt)

    zer = jnp.zeros((N_NODES, DH), jnp.float32)
    x0 = x[:, :DH]
    x1 = x[:, DH:]
    x0a, x0b = _segsum(x0, *idx, zer)
    x1a, x1b = _segsum(x1, *idx, zer)

    w1cat = jnp.transpose(W1, (1, 0, 2)).reshape(
        D_IN, N_EXPERTS * D_HID).astype(jnp.bfloat16)
    w2b = W2.astype(jnp.bfloat16)
    thr = gate_threshold.reshape(1, N_EXPERTS)

    u0, u1, u2, gates = _tc_experts(x, x0a, x0b, x1a, x1b, noise, w_gate,
                                    w_noise, thr, w1cat, w2b)

    p0a, p0b = _segsum(u0, *idx, zer)
    p1a, p1b = _segsum(u1, *idx, zer)
    p2a, p2b = _segsum(u2, *idx, zer)

    return _tc_combine(u0, u1, u2, p0a, p0b, p1a, p1b, p2a, p2b, gates)


# speedup vs baseline: 1.4053x; 1.0133x over previous
import functools

import jax
import jax.numpy as jnp
import numpy as np
from jax import lax
from jax.experimental import pallas as pl
from jax.experimental.pallas import tpu as pltpu
from jax.experimental.pallas import tpu_sc as plsc

N_NODES = 10000
N_EDGES = 160000
D_IN = 256
D_HID = 256
N_CLASSES = 40
N_EXPERTS = 8

NUM_SC_CORES = 2
NUM_SUBCORES = 16
EDGES_PER_TILE = N_EDGES // NUM_SUBCORES
EDGE_BATCH = 80
BATCHES_PER_TILE = EDGES_PER_TILE // EDGE_BATCH
ROWS_PER_TILE = 624
ROWS_LAST_TILE = N_NODES - ROWS_PER_TILE * (NUM_SUBCORES - 1)

EC = N_EXPERTS * N_CLASSES

NODE_BLOCK = 1000
NUM_NODE_BLOCKS = N_NODES // NODE_BLOCK


@functools.cache
def _mesh():
    return plsc.VectorSubcoreMesh(
        core_axis_name="c", subcore_axis_name="s",
        num_cores=NUM_SC_CORES, num_subcores=NUM_SUBCORES)


DH = 128


def _zero_my_rows(zer_hbm, acc_sh, s):
    row0 = s * ROWS_PER_TILE

    @pl.when(s < NUM_SUBCORES - 1)
    def _():
        pltpu.sync_copy(zer_hbm.at[pl.ds(row0, ROWS_PER_TILE)],
                        acc_sh.at[pl.ds(row0, ROWS_PER_TILE)])

    @pl.when(s == NUM_SUBCORES - 1)
    def _():
        pltpu.sync_copy(zer_hbm.at[pl.ds(row0, ROWS_LAST_TILE)],
                        acc_sh.at[pl.ds(row0, ROWS_LAST_TILE)])


def _writeback_my_rows(acc_sh, out, s):
    row0 = s * ROWS_PER_TILE

    @pl.when(s < NUM_SUBCORES - 1)
    def _():
        pltpu.sync_copy(acc_sh.at[pl.ds(row0, ROWS_PER_TILE)],
                        out.at[pl.ds(row0, ROWS_PER_TILE)])

    @pl.when(s == NUM_SUBCORES - 1)
    def _():
        pltpu.sync_copy(acc_sh.at[pl.ds(row0, ROWS_LAST_TILE)],
                        out.at[pl.ds(row0, ROWS_LAST_TILE)])


EDGES_PER_WORKER = N_EDGES // (NUM_SC_CORES * NUM_SUBCORES)
MAIN_BATCH = 88
N_MAIN = 56
TAIL_BATCH = EDGES_PER_WORKER - N_MAIN * MAIN_BATCH


@functools.cache
def _build_segsum():
    @functools.partial(
        pl.kernel,
        out_type=(
            jax.ShapeDtypeStruct((N_NODES, DH), jnp.float32),
            jax.ShapeDtypeStruct((N_NODES, DH), jnp.float32),
        ),
        mesh=_mesh(),
        scratch_types=[
            pltpu.VMEM((N_MAIN, MAIN_BATCH), jnp.int32),
            pltpu.VMEM((N_MAIN, MAIN_BATCH), jnp.int32),
            pltpu.VMEM((1, TAIL_BATCH), jnp.int32),
            pltpu.VMEM((1, TAIL_BATCH), jnp.int32),
            pltpu.VMEM((MAIN_BATCH, DH), jnp.float32),
            pltpu.VMEM((MAIN_BATCH, DH), jnp.float32),
            pltpu.VMEM((MAIN_BATCH, DH), jnp.float32),
            pltpu.VMEM_SHARED((N_NODES, DH), jnp.float32),
            pltpu.SemaphoreType.DMA,
            pltpu.SemaphoreType.DMA,
            pltpu.SemaphoreType.DMA,
            pltpu.SemaphoreType.DMA,
        ],
    )
    def segsum(h, srcm_hbm, dstm_hbm, srct_hbm, dstt_hbm, zer_hbm,
               outa, outb,
               srcm_v, dstm_v, srct_v, dstt_v, r0, r1, r2, acc_sh,
               g0, g1, g2, ss):
        c = lax.axis_index("c")
        s = lax.axis_index("s")
        bufs = (r0, r1, r2)
        gsems = (g0, g1, g2)

        pltpu.sync_copy(srcm_hbm.at[c, s], srcm_v)
        d1 = pltpu.async_copy(dstm_hbm.at[c, s], dstm_v, ss)
        d2 = pltpu.async_copy(srct_hbm.at[c, s], srct_v, ss)
        d3 = pltpu.async_copy(dstt_hbm.at[c, s], dstt_v, ss)

        pltpu.async_copy(h.at[srcm_v.at[0]], r0, g0)
        pltpu.async_copy(h.at[srcm_v.at[1]], r1, g1)
        _zero_my_rows(zer_hbm, acc_sh, s)
        d1.wait()
        d2.wait()
        d3.wait()
        plsc.subcore_barrier()

        def step(j, cur, gcur, prv, tw0):
            pltpu.make_async_copy(h.at[srcm_v.at[j]], cur, gcur).wait()

            @pl.when(j >= 1)
            def _():
                pltpu.make_async_copy(
                    prv, acc_sh.at[dstm_v.at[j - 1]], ss).wait()

            @pl.when(j < N_MAIN - 2)
            def _():
                pltpu.async_copy(h.at[srcm_v.at[j + 2]], prv, tw0)

            pltpu.async_copy(cur, acc_sh.at[dstm_v.at[j]], ss, add=True)

        def body(j, carry):
            @pl.when(j % 3 == 0)
            def _():
                step(j, r0, g0, r2, g2)

            @pl.when(j % 3 == 1)
            def _():
                step(j, r1, g1, r0, g0)

            @pl.when(j % 3 == 2)
            def _():
                step(j, r2, g2, r1, g1)

            return carry

        lax.fori_loop(0, N_MAIN, body, 0)

        pltpu.make_async_copy(
            bufs[(N_MAIN - 1) % 3],
            acc_sh.at[dstm_v.at[N_MAIN - 1]], ss).wait()
        rtail = r0.at[pl.ds(0, TAIL_BATCH)]
        pltpu.async_copy(h.at[srct_v.at[0]], rtail, g0).wait()
        pltpu.sync_copy(rtail, acc_sh.at[dstt_v.at[0]], add=True)
        plsc.subcore_barrier()

        @pl.when(c == 0)
        def _():
            _writeback_my_rows(acc_sh, outa, s)

        @pl.when(c == 1)
        def _():
            _writeback_my_rows(acc_sh, outb, s)

    return segsum


def _segsum(h, srcm, dstm, srct, dstt, zer):
    return _build_segsum()(h, srcm, dstm, srct, dstt, zer)


def _tc_experts_body(x_ref, a0a_ref, a0b_ref, a1a_ref, a1b_ref, noise_ref,
                     wg_ref, wn_ref, thr_ref, w1_ref, w2_ref,
                     u0_ref, u1_ref, u2_ref, gates_ref):
    x = x_ref[...]
    z = x + jnp.concatenate([a0a_ref[...] + a0b_ref[...],
                             a1a_ref[...] + a1b_ref[...]], axis=1)

    clean = jnp.dot(x, wg_ref[...], preferred_element_type=jnp.float32)
    t = jnp.dot(x, wn_ref[...], preferred_element_type=jnp.float32)
    std = jnp.log1p(jnp.exp(-jnp.abs(t))) + jnp.maximum(t, 0.0) + 1e-2
    noisy = clean + noise_ref[...] * std
    scores = noisy - thr_ref[...]
    open_mask = (scores > 0.0).astype(jnp.float32)
    m = jnp.max(noisy, axis=1, keepdims=True)
    ex = jnp.exp(noisy - m)
    sm = ex / jnp.sum(ex, axis=1, keepdims=True)
    raw = sm * open_mask
    gates_ref[...] = raw / (jnp.sum(raw, axis=1, keepdims=True) + 1e-9)

    h = jnp.maximum(
        jnp.dot(z.astype(jnp.bfloat16), w1_ref[...],
                preferred_element_type=jnp.float32), 0.0)
    hb = h.astype(jnp.bfloat16)
    us = [jnp.dot(hb[:, e * D_HID:(e + 1) * D_HID], w2_ref[e],
                  preferred_element_type=jnp.float32)
          for e in range(N_EXPERTS)]
    u = jnp.concatenate(
        us + [jnp.zeros((NODE_BLOCK, 3 * DH - EC), jnp.float32)], axis=1)
    u0_ref[...] = u[:, :DH]
    u1_ref[...] = u[:, DH:2 * DH]
    u2_ref[...] = u[:, 2 * DH:]


def _tc_experts(x, a0a, a0b, a1a, a1b, noise, w_gate, w_noise, thr,
                w1cat, w2bd):
    blk = lambda shape: pl.BlockSpec(shape, lambda i: (0, 0))
    return pl.pallas_call(
        _tc_experts_body,
        grid=(NUM_NODE_BLOCKS,),
        in_specs=[
            pl.BlockSpec((NODE_BLOCK, D_IN), lambda i: (i, 0)),
            pl.BlockSpec((NODE_BLOCK, DH), lambda i: (i, 0)),
            pl.BlockSpec((NODE_BLOCK, DH), lambda i: (i, 0)),
            pl.BlockSpec((NODE_BLOCK, DH), lambda i: (i, 0)),
            pl.BlockSpec((NODE_BLOCK, DH), lambda i: (i, 0)),
            pl.BlockSpec((NODE_BLOCK, N_EXPERTS), lambda i: (i, 0)),
            blk((D_IN, N_EXPERTS)),
            blk((D_IN, N_EXPERTS)),
            blk((1, N_EXPERTS)),
            blk((D_IN, N_EXPERTS * D_HID)),
            pl.BlockSpec((N_EXPERTS, D_HID, N_CLASSES), lambda i: (0, 0, 0)),
        ],
        out_specs=[
            pl.BlockSpec((NODE_BLOCK, DH), lambda i: (i, 0)),
            pl.BlockSpec((NODE_BLOCK, DH), lambda i: (i, 0)),
            pl.BlockSpec((NODE_BLOCK, DH), lambda i: (i, 0)),
            pl.BlockSpec((NODE_BLOCK, N_EXPERTS), lambda i: (i, 0)),
        ],
        out_shape=[
            jax.ShapeDtypeStruct((N_NODES, DH), jnp.float32),
            jax.ShapeDtypeStruct((N_NODES, DH), jnp.float32),
            jax.ShapeDtypeStruct((N_NODES, DH), jnp.float32),
            jax.ShapeDtypeStruct((N_NODES, N_EXPERTS), jnp.float32),
        ],
    )(x, a0a, a0b, a1a, a1b, noise, w_gate, w_noise, thr, w1cat, w2bd)


_REP_FULL = np.kron(np.eye(N_EXPERTS), np.ones((1, N_CLASSES))).astype(np.float32)
_SEL_FULL = np.kron(np.ones((N_EXPERTS, 1)), np.eye(N_CLASSES)).astype(np.float32)


_REP_PAD = np.concatenate(
    [_REP_FULL, np.zeros((N_EXPERTS, 3 * DH - EC), np.float32)], axis=1)
_SEL_PAD = np.concatenate(
    [_SEL_FULL, np.zeros((3 * DH - EC, N_CLASSES), np.float32)], axis=0)


def _tc_combine_body(u0_ref, u1_ref, u2_ref, p0a_ref, p0b_ref, p1a_ref,
                     p1b_ref, p2a_ref, p2b_ref, gates_ref, repp_ref,
                     selp_ref, y_ref):
    g = gates_ref[...]
    su = jnp.concatenate(
        [u0_ref[...] + p0a_ref[...] + p0b_ref[...],
         u1_ref[...] + p1a_ref[...] + p1b_ref[...],
         u2_ref[...] + p2a_ref[...] + p2b_ref[...]], axis=1)
    gp = jnp.dot(g, repp_ref[...], preferred_element_type=jnp.float32)
    y_ref[...] = jnp.dot(su * gp, selp_ref[...],
                         preferred_element_type=jnp.float32)


def _tc_combine(u0, u1, u2, p0a, p0b, p1a, p1b, p2a, p2b, gates):
    blk = lambda shape: pl.BlockSpec(shape, lambda i: (0, 0))
    consts = (jnp.asarray(_REP_PAD), jnp.asarray(_SEL_PAD))
    nb = lambda: pl.BlockSpec((NODE_BLOCK, DH), lambda i: (i, 0))
    return pl.pallas_call(
        _tc_combine_body,
        grid=(NUM_NODE_BLOCKS,),
        in_specs=[
            nb(), nb(), nb(), nb(), nb(), nb(), nb(), nb(), nb(),
            pl.BlockSpec((NODE_BLOCK, N_EXPERTS), lambda i: (i, 0)),
            blk((N_EXPERTS, 3 * DH)), blk((3 * DH, N_CLASSES)),
        ],
        out_specs=pl.BlockSpec((NODE_BLOCK, N_CLASSES), lambda i: (i, 0)),
        out_shape=jax.ShapeDtypeStruct((N_NODES, N_CLASSES), jnp.float32),
    )(u0, u1, u2, p0a, p0b, p1a, p1b, p2a, p2b, gates, *consts)


def kernel(x, edge_index, noise, w_gate, w_noise, gate_threshold, W1, W2):
    ncut = N_MAIN * MAIN_BATCH
    src = edge_index[0].astype(jnp.int32).reshape(
        NUM_SC_CORES, NUM_SUBCORES, EDGES_PER_WORKER)
    dst = edge_index[1].astype(jnp.int32).reshape(
        NUM_SC_CORES, NUM_SUBCORES, EDGES_PER_WORKER)
    srcm = src[:, :, :ncut].reshape(NUM_SC_CORES, NUM_SUBCORES,
                                    N_MAIN, MAIN_BATCH)
    dstm = dst[:, :, :ncut].reshape(NUM_SC_CORES, NUM_SUBCORES,
                                    N_MAIN, MAIN_BATCH)
    srct = src[:, :, ncut:].reshape(NUM_SC_CORES, NUM_SUBCORES,
                                    1, TAIL_BATCH)
    dstt = dst[:, :, ncut:].reshape(NUM_SC_CORES, NUM_SUBCORES,
                                    1, TAIL_BATCH)
    idx = (srcm, dstm, srct, dstt)

    zer = jnp.zeros((N_NODES, DH), jnp.float32)
    x0 = x[:, :DH]
    x1 = x[:, DH:]
    x0a, x0b = _segsum(x0, *idx, zer)
    x1a, x1b = _segsum(x1, *idx, zer)

    w1cat = jnp.transpose(W1, (1, 0, 2)).reshape(
        D_IN, N_EXPERTS * D_HID).astype(jnp.bfloat16)
    w2b = W2.astype(jnp.bfloat16)
    thr = gate_threshold.reshape(1, N_EXPERTS)

    u0, u1, u2, gates = _tc_experts(x, x0a, x0b, x1a, x1b, noise, w_gate,
                                    w_noise, thr, w1cat, w2b)

    p0a, p0b = _segsum(u0, *idx, zer)
    p1a, p1b = _segsum(u1, *idx, zer)
    p2a, p2b = _segsum(u2, *idx, zer)

    return _tc_combine(u0, u1, u2, p0a, p0b, p1a, p1b, p2a, p2b, gates)
